# Initial kernel scaffold; baseline (speedup 1.0000x reference)
import functools

import jax
import jax.numpy as jnp
from jax.experimental import pallas as pl
from jax.experimental.pallas import tpu as pltpu

B = 128
I = 512
O = 512
W = 512
K = 154
WB = 8


def _moe_body(x_ref, dw_ref, tiles_ref, out_ref):
    i = pl.program_id(0)

    @pl.when(i == 0)
    def _init():
        out_ref[...] = jnp.zeros_like(out_ref)

    x = x_ref[...]
    acc = out_ref[...]
    for j in range(WB):
        t = tiles_ref[j]
        p = jax.lax.dot_general(
            x, t, (((1,), (1,)), ((), ())), preferred_element_type=jnp.float32
        )
        wcol = dw_ref[:, pl.ds(j, 1)]
        acc = acc + wcol * p
    out_ref[...] = acc


def kernel(x, gate_w, gate_b, tiles):
    logits = x @ gate_w.T + gate_b
    scores, idx = jax.lax.top_k(logits, K)
    wts = jax.nn.softmax(scores, axis=-1)
    dense = jnp.zeros((B, W), jnp.float32)
    dense = dense.at[jnp.arange(B)[:, None], idx].set(wts)

    grid = W // WB
    out = pl.pallas_call(
        _moe_body,
        grid=(grid,),
        in_specs=[
            pl.BlockSpec((B, I), lambda i: (0, 0)),
            pl.BlockSpec((B, WB), lambda i: (0, i)),
            pl.BlockSpec((WB, O, I), lambda i: (i, 0, 0)),
        ],
        out_specs=pl.BlockSpec((B, O), lambda i: (0, 0)),
        out_shape=jax.ShapeDtypeStruct((B, O), jnp.float32),
    )(x, dense, tiles)
    return out


# speedup vs baseline: 1.6356x; 1.6356x over previous
import jax
import jax.numpy as jnp
from jax.experimental import pallas as pl

B = 128
I = 512
O = 512
W = 512
K = 154
WB = 8


def _moe_body(x_ref, dwt_ref, tiles_ref, outt_ref):
    i = pl.program_id(0)

    @pl.when(i == 0)
    def _init():
        outt_ref[...] = jnp.zeros_like(outt_ref)

    x = x_ref[...]
    dwb = dwt_ref[0]
    acc = outt_ref[...]
    for j in range(WB):
        t = tiles_ref[j]
        pt = jax.lax.dot_general(
            t, x, (((1,), (1,)), ((), ())), preferred_element_type=jnp.float32
        )
        acc = acc + dwb[j : j + 1, :] * pt
    outt_ref[...] = acc


def kernel(x, gate_w, gate_b, tiles):
    logits = x @ gate_w.T + gate_b
    scores, idx = jax.lax.top_k(logits, K)
    wts = jax.nn.softmax(scores, axis=-1)
    dense_t = jnp.zeros((W, B), jnp.float32)
    dense_t = dense_t.at[idx, jnp.arange(B)[:, None]].set(wts)
    dwt3 = dense_t.reshape(W // WB, WB, B)

    grid = W // WB
    outt = pl.pallas_call(
        _moe_body,
        grid=(grid,),
        in_specs=[
            pl.BlockSpec((B, I), lambda i: (0, 0)),
            pl.BlockSpec((1, WB, B), lambda i: (i, 0, 0)),
            pl.BlockSpec((WB, O, I), lambda i: (i, 0, 0)),
        ],
        out_specs=pl.BlockSpec((O, B), lambda i: (0, 0)),
        out_shape=jax.ShapeDtypeStruct((O, B), jnp.float32),
    )(x, dwt3, tiles)
    return outt.T


# speedup vs baseline: 2.3989x; 1.4667x over previous
import jax
import jax.numpy as jnp
from jax.experimental import pallas as pl

B = 128
I = 512
O = 512
W = 512
K = 154
WB = 8


def _route_body(x_ref, gw_ref, gb_ref, dwt_ref):
    logits = jax.lax.dot_general(
        gw_ref[...], x_ref[...], (((1,), (1,)), ((), ())),
        preferred_element_type=jnp.float32,
    ) + gb_ref[...]

    bits = jax.lax.bitcast_convert_type(logits, jnp.uint32)
    sign = bits >> jnp.uint32(31)
    key = jnp.where(sign == jnp.uint32(1), ~bits, bits | jnp.uint32(0x80000000))

    lo = jnp.zeros((1, B), jnp.uint32)
    hi = jnp.full((1, B), jnp.uint32(0xFFFFFFFF))

    def body(_, carry):
        lo, hi = carry
        mid = lo + ((hi - lo) >> jnp.uint32(1))
        cnt = jnp.sum((key >= mid).astype(jnp.int32), axis=0, keepdims=True)
        ge = cnt >= K
        return jnp.where(ge, mid, lo), jnp.where(ge, hi, mid)

    lo, hi = jax.lax.fori_loop(0, 32, body, (lo, hi))

    mask = key >= lo
    m = jnp.max(logits, axis=0, keepdims=True)
    e = jnp.where(mask, jnp.exp(logits - m), 0.0)
    denom = jnp.sum(e, axis=0, keepdims=True)
    dwt_ref[...] = e / denom


def _moe_body(x_ref, dwt_ref, tiles_ref, outt_ref):
    i = pl.program_id(0)

    @pl.when(i == 0)
    def _init():
        outt_ref[...] = jnp.zeros_like(outt_ref)

    x = x_ref[...]
    dwb = dwt_ref[0]
    acc = outt_ref[...]
    for j in range(WB):
        t = tiles_ref[j]
        pt = jax.lax.dot_general(
            t, x, (((1,), (1,)), ((), ())), preferred_element_type=jnp.float32
        )
        acc = acc + dwb[j : j + 1, :] * pt
    outt_ref[...] = acc


def kernel(x, gate_w, gate_b, tiles):
    gb2 = jnp.broadcast_to(gate_b[:, None], (W, B))

    dwt = pl.pallas_call(
        _route_body,
        in_specs=[
            pl.BlockSpec((B, I), lambda: (0, 0)),
            pl.BlockSpec((W, I), lambda: (0, 0)),
            pl.BlockSpec((W, B), lambda: (0, 0)),
        ],
        out_specs=pl.BlockSpec((W, B), lambda: (0, 0)),
        out_shape=jax.ShapeDtypeStruct((W, B), jnp.float32),
    )(x, gate_w, gb2)

    dwt3 = dwt.reshape(W // WB, WB, B)

    grid = W // WB
    outt = pl.pallas_call(
        _moe_body,
        grid=(grid,),
        in_specs=[
            pl.BlockSpec((B, I), lambda i: (0, 0)),
            pl.BlockSpec((1, WB, B), lambda i: (i, 0, 0)),
            pl.BlockSpec((WB, O, I), lambda i: (i, 0, 0)),
        ],
        out_specs=pl.BlockSpec((O, B), lambda i: (0, 0)),
        out_shape=jax.ShapeDtypeStruct((O, B), jnp.float32),
    )(x, dwt3, tiles)
    return outt.T
